# baseline (speedup 1.0000x reference)
import functools

import jax
import jax.numpy as jnp
from jax import lax
from jax.experimental import pallas as pl
from jax.experimental.pallas import tpu as pltpu

NSEM = 8


@functools.lru_cache(maxsize=None)
def _make_tc(batch: int, num_nodes: int, d_emb: int):
    per_sem = batch // NSEM

    def body(ids_smem, tgt_ref, ctx_ref, out_t, out_c, *sems):
        for tbl, out in ((tgt_ref, out_t), (ctx_ref, out_c)):

            def loop(i, _):
                for q in range(NSEM):
                    row = ids_smem[i * NSEM + q]
                    pltpu.make_async_copy(
                        tbl.at[row], out.at[i * NSEM + q], sems[q]).start()
                return 0

            lax.fori_loop(0, per_sem, loop, 0, unroll=2)
            for q in range(NSEM):
                pltpu.make_async_copy(
                    tbl.at[pl.ds(0, per_sem)],
                    out.at[pl.ds(0, per_sem)], sems[q]).wait()

    return pl.pallas_call(
        body,
        grid_spec=pltpu.PrefetchScalarGridSpec(
            num_scalar_prefetch=1,
            grid=(1,),
            in_specs=[pl.BlockSpec(memory_space=pltpu.MemorySpace.HBM)] * 2,
            out_specs=[pl.BlockSpec(memory_space=pltpu.MemorySpace.HBM)] * 2,
            scratch_shapes=[pltpu.SemaphoreType.DMA] * NSEM,
        ),
        out_shape=(
            jax.ShapeDtypeStruct((batch, d_emb), jnp.float32),
            jax.ShapeDtypeStruct((batch, d_emb), jnp.float32),
        ),
    )


def kernel(ids, E_target, E_context):
    ids = ids.astype(jnp.int32)
    n, d = E_target.shape
    k = _make_tc(ids.shape[0], n, d)
    return k(ids, E_target, E_context)


# speedup vs baseline: 1.6535x; 1.6535x over previous
import functools

import jax
import jax.numpy as jnp
from jax import lax
from jax.experimental import pallas as pl
from jax.experimental.pallas import tpu as pltpu
from jax.experimental.pallas import tpu_sc as plsc

LANES = 16
NSEM = 8


@functools.lru_cache(maxsize=None)
def _make(batch: int, num_nodes: int, d_emb: int):
    info = plsc.get_sparse_core_info()
    nc, ns = info.num_cores, info.num_subcores
    nw = nc * ns
    b_per_w = batch // nw
    rows_per_sem = b_per_w // NSEM
    mesh = plsc.VectorSubcoreMesh(core_axis_name="c", subcore_axis_name="s")

    @functools.partial(
        pl.kernel,
        mesh=mesh,
        out_type=(
            jax.ShapeDtypeStruct((batch, d_emb), jnp.float32),
            jax.ShapeDtypeStruct((batch, d_emb), jnp.float32),
        ),
        scratch_types=[
            pltpu.VMEM((b_per_w + LANES,), jnp.int32),
            pltpu.VMEM((b_per_w, d_emb), jnp.float32),
        ] + [pltpu.SemaphoreType.DMA] * NSEM,
        compiler_params=pltpu.CompilerParams(use_tc_tiling_on_sc=True),
    )
    def k(ids_hbm, tgt_hbm, ctx_hbm, out_t, out_c, idx_v, rows_v, *sems):
        wid = lax.axis_index("s") * nc + lax.axis_index("c")
        base = wid * b_per_w
        pltpu.sync_copy(ids_hbm.at[pl.ds(base, b_per_w)],
                        idx_v.at[pl.ds(0, b_per_w)])
        out_slice = pl.ds(base, b_per_w)
        for tbl, out_hbm in ((tgt_hbm, out_t), (ctx_hbm, out_c)):

            def body(i, _):
                for q in range(NSEM):
                    row = idx_v[pl.ds(i * NSEM + q, LANES)][0]
                    pltpu.async_copy(tbl.at[row], rows_v.at[i * NSEM + q],
                                     sems[q])
                return 0

            lax.fori_loop(0, rows_per_sem, body, 0)
            for q in range(NSEM):
                pltpu.make_async_copy(
                    out_hbm.at[pl.ds(base, rows_per_sem)],
                    rows_v.at[pl.ds(0, rows_per_sem)], sems[q]).wait()
            pltpu.sync_copy(rows_v, out_hbm.at[out_slice])

    return k


def kernel(ids, E_target, E_context):
    ids = ids.astype(jnp.int32)
    n, d = E_target.shape
    k = _make(ids.shape[0], n, d)
    return k(ids, E_target, E_context)
